# Initial kernel scaffold; baseline (speedup 1.0000x reference)
import jax
import jax.numpy as jnp
from jax.experimental import pallas as pl


def kernel(x, xe, edge_index, edge_type, weight, root, bias):
    raise NotImplementedError("write your pallas kernel here")


# speedup vs baseline: 2.0737x; 2.0737x over previous
import functools
import jax
import jax.numpy as jnp
from jax import lax
from jax.experimental import pallas as pl
from jax.experimental.pallas import tpu as pltpu
from jax.experimental.pallas import tpu_sc as plsc

NREL = 4
N = 10000
E = 320000
D = 128
ROWS = 10240
DUMMY = 10100
NC, NS = 2, 16
NW = NC * NS
EPAD = 327680
EPW = EPAD // NW
CH = 128
NCH = EPW // CH
BE = 1024
GA = EPAD // BE
BN = 512
GB = ROWS // BN

_mesh = plsc.VectorSubcoreMesh(core_axis_name="c", subcore_axis_name="s")


@functools.partial(
    pl.kernel, mesh=_mesh,
    out_type=jax.ShapeDtypeStruct((EPAD, D), jnp.float32),
    scratch_types=[
        pltpu.VMEM((CH,), jnp.int32),
        pltpu.VMEM((CH, D), jnp.float32),
        pltpu.SemaphoreType.DMA,
    ],
)
def _sc_gather(x_hbm, src_hbm, out_hbm, idx_v, rows_v, sem):
  wid = lax.axis_index("s") * NC + lax.axis_index("c")
  base = wid * EPW

  @pl.loop(0, NCH)
  def _chunk(j):
    off = base + j * CH
    pltpu.sync_copy(src_hbm.at[pl.ds(off, CH)], idx_v)
    pltpu.async_copy(x_hbm.at[idx_v], rows_v, sem).wait()
    pltpu.sync_copy(rows_v, out_hbm.at[pl.ds(off, CH)])


@functools.partial(
    pl.kernel, mesh=_mesh,
    out_type=(jax.ShapeDtypeStruct((NC * NREL, ROWS, D), jnp.float32),
              jax.ShapeDtypeStruct((NC * NREL, ROWS, D), jnp.float32)),
    scratch_types=[
        pltpu.VMEM((CH,), jnp.int32),
        pltpu.VMEM((CH, D), jnp.float32),
        pltpu.VMEM((CH, D), jnp.float32),
        pltpu.VMEM_SHARED((ROWS, D), jnp.float32),
    ],
)
def _sc_scatter(msg_hbm, idx_hbm, zacc_hbm, ones_hbm,
                acc_out, cnt_out, idx_v, rows_v, ones_v, acc_sh):
  cid = lax.axis_index("c")
  sid = lax.axis_index("s")
  wid = sid * NC + cid
  base = wid * EPW
  pltpu.sync_copy(ones_hbm, ones_v)

  for r in range(NREL):
    @pl.when(sid == 0)
    def _zero():
      pltpu.sync_copy(zacc_hbm, acc_sh)
    plsc.subcore_barrier()

    @pl.loop(0, NCH)
    def _msgchunk(j):
      off = base + j * CH
      pltpu.sync_copy(idx_hbm.at[pl.ds(r * EPAD + off, CH)], idx_v)
      pltpu.sync_copy(msg_hbm.at[pl.ds(off, CH)], rows_v)
      pltpu.sync_copy(rows_v, acc_sh.at[idx_v], add=True)

    plsc.subcore_barrier()

    @pl.when(sid == 0)
    def _flush_zero():
      pltpu.sync_copy(acc_sh, acc_out.at[cid * NREL + r])
      pltpu.sync_copy(zacc_hbm, acc_sh)
    plsc.subcore_barrier()

    @pl.loop(0, NCH)
    def _cntchunk(j):
      off = base + j * CH
      pltpu.sync_copy(idx_hbm.at[pl.ds(r * EPAD + off, CH)], idx_v)
      pltpu.sync_copy(ones_v, acc_sh.at[idx_v], add=True)

    plsc.subcore_barrier()

    @pl.when(sid == 0)
    def _flush_cnt():
      pltpu.sync_copy(acc_sh, cnt_out.at[cid * NREL + r])


def _msg_body(xs_ref, xe_ref, dst_ref, typ_ref,
              msg_ref, i0_ref, i1_ref, i2_ref, i3_ref):
  msg_ref[...] = jnp.maximum(xs_ref[...] + xe_ref[...], 0.0)
  d = dst_ref[0, 0, :]
  t = typ_ref[0, 0, :]
  for r, ref in enumerate((i0_ref, i1_ref, i2_ref, i3_ref)):
    ref[0, 0, :] = jnp.where(t == r, d, DUMMY)


def _tc_msg(xs, xe_p, dst3, typ3):
  ispec = pl.BlockSpec((1, 1, BE), lambda i: (i, 0, 0))
  espec = pl.BlockSpec((BE, D), lambda i: (i, 0))
  return pl.pallas_call(
      _msg_body,
      grid=(GA,),
      in_specs=[espec, espec, ispec, ispec],
      out_specs=[espec, ispec, ispec, ispec, ispec],
      out_shape=[
          jax.ShapeDtypeStruct((EPAD, D), jnp.float32),
          jax.ShapeDtypeStruct((GA, 1, BE), jnp.int32),
          jax.ShapeDtypeStruct((GA, 1, BE), jnp.int32),
          jax.ShapeDtypeStruct((GA, 1, BE), jnp.int32),
          jax.ShapeDtypeStruct((GA, 1, BE), jnp.int32),
      ],
  )(xs, xe_p, dst3, typ3)


def _fin_body(acc_ref, scale_ref, x_ref, w_ref, root_ref, bias_ref, out_ref):
  acc = acc_ref[...]
  h = (acc[0] + acc[1]) * scale_ref[...]
  o = jnp.dot(x_ref[...], root_ref[...], preferred_element_type=jnp.float32)
  for r in range(NREL):
    o = o + jnp.dot(h[r], w_ref[r], preferred_element_type=jnp.float32)
  out_ref[...] = o + bias_ref[...]


def _tc_fin(acc, scale3, x_p, weight, root, bias2):
  return pl.pallas_call(
      _fin_body,
      grid=(GB,),
      in_specs=[
          pl.BlockSpec((NC, NREL, BN, D), lambda i: (0, 0, i, 0)),
          pl.BlockSpec((NREL, BN, D), lambda i: (0, i, 0)),
          pl.BlockSpec((BN, D), lambda i: (i, 0)),
          pl.BlockSpec((NREL, D, D), lambda i: (0, 0, 0)),
          pl.BlockSpec((D, D), lambda i: (0, 0)),
          pl.BlockSpec((1, D), lambda i: (0, 0)),
      ],
      out_specs=pl.BlockSpec((BN, D), lambda i: (i, 0)),
      out_shape=jax.ShapeDtypeStruct((ROWS, D), jnp.float32),
  )(acc, scale3, x_p, weight, root, bias2)


def kernel(x, xe, edge_index, edge_type, weight, root, bias):
  src = edge_index[0].astype(jnp.int32)
  dst = edge_index[1].astype(jnp.int32)
  typ = edge_type.astype(jnp.int32)
  pe = EPAD - E
  src_p = jnp.pad(src, (0, pe))
  dst_p = jnp.pad(dst, (0, pe))
  typ_p = jnp.pad(typ, (0, pe), constant_values=NREL)
  xe_p = jnp.pad(xe, ((0, pe), (0, 0)))

  xs = _sc_gather(x, src_p)
  msg, i0, i1, i2, i3 = _tc_msg(xs, xe_p,
                                dst_p.reshape(GA, 1, BE),
                                typ_p.reshape(GA, 1, BE))
  idx4 = jnp.concatenate([i0.reshape(EPAD), i1.reshape(EPAD),
                          i2.reshape(EPAD), i3.reshape(EPAD)])

  zacc = jnp.zeros((ROWS, D), jnp.float32)
  ones = jnp.ones((CH, D), jnp.float32)
  acc4, cnt4 = _sc_scatter(msg, idx4, zacc, ones)
  acc = acc4.reshape(NC, NREL, ROWS, D)
  cnt = cnt4.reshape(NC, NREL, ROWS, D)

  c = cnt[0, :, :, 0] + cnt[1, :, :, 0]
  scale = 1.0 / jnp.clip(c, 1.0, None)
  scale3 = jnp.broadcast_to(scale[:, :, None], (NREL, ROWS, D))
  x_p = jnp.pad(x, ((0, ROWS - N), (0, 0)))
  out_p = _tc_fin(acc, scale3, x_p, weight, root, bias.reshape(1, D))
  return out_p[:N]
